# baseline (speedup 1.0000x reference)
import jax
import jax.numpy as jnp
from jax.experimental import pallas as pl
from jax.experimental.pallas import tpu as pltpu

_NO = 8
_R = 2
_G = 8


def _dma_kernel(tid_ref, ep_ref, *refs):
    outs = refs[:_NO]
    sel_buf = refs[_NO]
    bufs = refs[_NO + 1:_NO + 1 + _NO]
    gsem = refs[_NO + 1 + _NO]
    sems = refs[_NO + 2 + _NO:]
    tid = tid_ref[0]
    cp = pltpu.make_async_copy(ep_ref.at[:, tid], sel_buf, gsem)
    cp.start()
    cp.wait()
    src = sel_buf[...][:, None]
    for k in range(_NO):
        bufs[k][...] = jnp.broadcast_to(src, bufs[k].shape)
    for k in range(_NO):
        for g in range(_G):
            pltpu.make_async_copy(
                bufs[k], outs[k].at[:, pl.ds(g * _R, _R)], sems[k]
            ).start()
    for k in range(_NO):
        for g in range(_G):
            pltpu.make_async_copy(
                bufs[k], outs[k].at[:, pl.ds(g * _R, _R)], sems[k]
            ).wait()


def kernel(x_query, vis_mark, e_p, task_id):
    del vis_mark
    B = x_query.shape[0]
    nL, _, P, D = e_p.shape
    tid = jnp.asarray(task_id, jnp.int32).reshape((1,))
    scratch = [pltpu.VMEM((nL, P, D), jnp.float32)]
    scratch += [pltpu.VMEM((nL, _R, P, D), jnp.float32) for _ in range(_NO)]
    scratch += [pltpu.SemaphoreType.DMA]
    scratch += [pltpu.SemaphoreType.DMA for _ in range(_NO)]
    return pl.pallas_call(
        _dma_kernel,
        grid_spec=pltpu.PrefetchScalarGridSpec(
            num_scalar_prefetch=1,
            grid=(1,),
            in_specs=[pl.BlockSpec(memory_space=pl.ANY)],
            out_specs=[pl.BlockSpec(memory_space=pl.ANY) for _ in range(_NO)],
            scratch_shapes=scratch,
        ),
        out_shape=[
            jax.ShapeDtypeStruct((nL, _R * _G, P, D), e_p.dtype)
            for _ in range(_NO)
        ],
    )(tid, e_p)


# speedup vs baseline: 1.2797x; 1.2797x over previous
import jax
import jax.numpy as jnp
from jax.experimental import pallas as pl
from jax.experimental.pallas import tpu as pltpu

_R = 16


def _dma_kernel(tid_ref, ep_ref, out_ref, sel_buf, big_buf, gsem, wsem):
    nL, B = out_ref.shape[0], out_ref.shape[1]
    groups = B // _R
    tid = tid_ref[0]
    gcp = pltpu.make_async_copy(ep_ref.at[:, tid], sel_buf, gsem)
    gcp.start()
    gcp.wait()
    src = sel_buf[...][:, None]
    big_buf[...] = jnp.broadcast_to(src, big_buf.shape)
    for l in range(nL):
        for g in range(groups):
            pltpu.make_async_copy(
                big_buf.at[pl.ds(l, 1)],
                out_ref.at[pl.ds(l, 1), pl.ds(g * _R, _R)],
                wsem,
            ).start(priority=(l * groups + g) % 2)
    for l in range(nL):
        for g in range(groups):
            pltpu.make_async_copy(
                big_buf.at[pl.ds(l, 1)],
                out_ref.at[pl.ds(l, 1), pl.ds(g * _R, _R)],
                wsem,
            ).wait()


def kernel(x_query, vis_mark, e_p, task_id):
    del vis_mark
    B = x_query.shape[0]
    nL, _, P, D = e_p.shape
    tid = jnp.asarray(task_id, jnp.int32).reshape((1,))
    return pl.pallas_call(
        _dma_kernel,
        grid_spec=pltpu.PrefetchScalarGridSpec(
            num_scalar_prefetch=1,
            grid=(1,),
            in_specs=[pl.BlockSpec(memory_space=pl.ANY)],
            out_specs=pl.BlockSpec(memory_space=pl.ANY),
            scratch_shapes=[
                pltpu.VMEM((nL, P, D), jnp.float32),
                pltpu.VMEM((nL, _R, P, D), jnp.float32),
                pltpu.SemaphoreType.DMA,
                pltpu.SemaphoreType.DMA,
            ],
        ),
        out_shape=jax.ShapeDtypeStruct((nL, B, P, D), e_p.dtype),
    )(tid, e_p)
